# baseline (speedup 1.0000x reference)
import jax
import jax.numpy as jnp
from jax import lax
from jax.experimental import pallas as pl
from jax.experimental.pallas import tpu as pltpu
from jax.experimental.pallas import tpu_sc as plsc

N = 10000
E = 320000
F_IN = 128
HID = 64
C = 16
K_HOPS = 10
ALPHA = 0.1

NS = 16
EPT = E // NS
CHE = 125
NCH = EPT // CHE
NP = 10112
RPT = NP // NS


def _mlp_body(x_ref, w1_ref, w2_ref, o_ref):
    h = lax.dot_general(x_ref[...], w1_ref[...], (((1,), (1,)), ((), ())),
                        preferred_element_type=jnp.float32)
    h = jnp.maximum(h, 0.0)
    o_ref[...] = lax.dot_general(h, w2_ref[...], (((1,), (1,)), ((), ())),
                                 preferred_element_type=jnp.float32)


def _mlp(x, W1, W2):
    blk = 2000
    return pl.pallas_call(
        _mlp_body,
        grid=(N // blk,),
        in_specs=[
            pl.BlockSpec((blk, F_IN), lambda i: (i, 0)),
            pl.BlockSpec((HID, F_IN), lambda i: (0, 0)),
            pl.BlockSpec((C, HID), lambda i: (0, 0)),
        ],
        out_specs=pl.BlockSpec((blk, C), lambda i: (i, 0)),
        out_shape=jax.ShapeDtypeStruct((N, C), jnp.float32),
    )(x, W1, W2)


def _lsm_body(h_ref, o_ref):
    h = h_ref[...]
    m = jnp.max(h, axis=1, keepdims=True)
    e = jnp.exp(h - m)
    lse = jnp.log(jnp.sum(e, axis=1, keepdims=True))
    o_ref[...] = h - m - lse


def _log_softmax(h):
    blk = 2000
    return pl.pallas_call(
        _lsm_body,
        grid=(N // blk,),
        in_specs=[pl.BlockSpec((blk, C), lambda i: (i, 0))],
        out_specs=pl.BlockSpec((blk, C), lambda i: (i, 0)),
        out_shape=jax.ShapeDtypeStruct((N, C), jnp.float32),
    )(h)


def _rsqrt16(d):
    t = 0.5 * (1.0 + d)
    for _ in range(9):
        t = 0.5 * (t + d / t)
    return 1.0 / t


ZCH = RPT // 8


def _appnp_body(out_hbm, src_hbm, dst_hbm, h_hbm,
                g_sh, acc_sh,
                src_v, dst_v, rb0, rb1, rb2, rb3, onesb, zb, abuf,
                c1_t, c2_t, dinv_t, g_t,
                gs0, gs1, gs2, gs3, ss0, ss1, ss2, ss3):
    rowbufs = (rb0, rb1, rb2, rb3)
    gsems = (gs0, gs1, gs2, gs3)
    ssems = (ss0, ss1, ss2, ss3)
    sid = lax.axis_index("s")
    cid = lax.axis_index("c")
    rbase = sid * RPT
    rows = pl.ds(rbase, RPT)

    pltpu.sync_copy(src_hbm.at[sid], src_v)
    pltpu.sync_copy(dst_hbm.at[sid], dst_v)

    def _fill1(r, _):
        onesb[r] = jnp.full((C,), 1.0, jnp.float32)
        return _
    lax.fori_loop(0, CHE, _fill1, 0)

    def _fill0(r, _):
        zb[r] = jnp.zeros((C,), jnp.float32)
        return _
    lax.fori_loop(0, ZCH, _fill0, 0)

    for z in range(8):
        pltpu.sync_copy(onesb.at[pl.ds(0, ZCH)],
                        acc_sh.at[pl.ds(rbase + z * ZCH, ZCH)])
    plsc.subcore_barrier()

    def _deg(j, _):
        pltpu.sync_copy(onesb, acc_sh.at[dst_v.at[j]], add=True)
        return _
    lax.fori_loop(0, NCH, _deg, 0)
    plsc.subcore_barrier()

    pltpu.sync_copy(acc_sh.at[rows], abuf)

    def _setup1(r, _):
        y = _rsqrt16(abuf[r])
        dinv_t[r] = y
        c1_t[r] = (1.0 - ALPHA) * y * y
        return _
    lax.fori_loop(0, RPT, _setup1, 0)
    for z in range(8):
        pltpu.sync_copy(zb, acc_sh.at[pl.ds(rbase + z * ZCH, ZCH)])

    pltpu.sync_copy(out_hbm.at[rows], abuf)

    def _setup2(r, _):
        g0 = dinv_t[r] * abuf[r]
        g_t[r] = g0
        c2_t[r] = ALPHA * g0
        return _
    lax.fori_loop(0, RPT, _setup2, 0)
    pltpu.sync_copy(g_t, g_sh.at[rows])
    plsc.subcore_barrier()

    nb = len(rowbufs)

    def _hop(k, _):
        for X in range(nb):
            pltpu.async_copy(g_sh.at[src_v.at[X]], rowbufs[X], gsems[X])

        def _ring(i, _):
            j0 = nb * i
            for X in range(nb):
                pltpu.make_async_copy(g_sh.at[src_v.at[j0 + X]],
                                      rowbufs[X], gsems[X]).wait()
                pltpu.async_copy(rowbufs[X], acc_sh.at[dst_v.at[j0 + X]],
                                 ssems[X], add=True)
            for X in range(nb):
                jn = jnp.minimum(j0 + nb + X, NCH - 1)
                pltpu.make_async_copy(rowbufs[X], acc_sh.at[dst_v.at[j0 + X]],
                                      ssems[X]).wait()
                pltpu.async_copy(g_sh.at[src_v.at[jn]], rowbufs[X], gsems[X])
            return _
        lax.fori_loop(0, NCH // nb, _ring, 0)
        for X in range(nb):
            pltpu.make_async_copy(g_sh.at[src_v.at[0]],
                                  rowbufs[X], gsems[X]).wait()
        plsc.subcore_barrier()

        pltpu.sync_copy(acc_sh.at[rows], abuf)
        for z in range(8):
            pltpu.sync_copy(zb, acc_sh.at[pl.ds(rbase + z * ZCH, ZCH)])

        def _upd(r, _):
            g_t[r] = c1_t[r] * (abuf[r] + g_t[r]) + c2_t[r]
            return _
        lax.fori_loop(0, RPT, _upd, 0)
        pltpu.sync_copy(g_t, g_sh.at[rows])
        plsc.subcore_barrier()
        return _
    lax.fori_loop(0, K_HOPS, _hop, 0)

    @pl.when(cid == 0)
    def _emit():
        def _div(r, _):
            abuf[r] = g_t[r] / dinv_t[r]
            return _
        lax.fori_loop(0, RPT, _div, 0)
        pltpu.sync_copy(abuf, h_hbm.at[rows])


def _appnp(out, src, dst):
    mesh = plsc.VectorSubcoreMesh(core_axis_name="c", subcore_axis_name="s")
    f = pl.kernel(
        _appnp_body,
        out_type=jax.ShapeDtypeStruct((NP, C), jnp.float32),
        mesh=mesh,
        compiler_params=pltpu.CompilerParams(use_tc_tiling_on_sc=False),
        scratch_types=[
            pltpu.VMEM_SHARED((NP, C), jnp.float32),
            pltpu.VMEM_SHARED((NP, C), jnp.float32),
            pltpu.VMEM((NCH, CHE), jnp.int32),
            pltpu.VMEM((NCH, CHE), jnp.int32),
            pltpu.VMEM((CHE, C), jnp.float32),
            pltpu.VMEM((CHE, C), jnp.float32),
            pltpu.VMEM((CHE, C), jnp.float32),
            pltpu.VMEM((CHE, C), jnp.float32),
            pltpu.VMEM((CHE, C), jnp.float32),
            pltpu.VMEM((ZCH, C), jnp.float32),
            pltpu.VMEM((RPT, C), jnp.float32),
            pltpu.VMEM((RPT, C), jnp.float32),
            pltpu.VMEM((RPT, C), jnp.float32),
            pltpu.VMEM((RPT, C), jnp.float32),
            pltpu.VMEM((RPT, C), jnp.float32),
        ] + [pltpu.SemaphoreType.DMA] * 8,
    )
    out_p = jnp.pad(out, ((0, NP - N), (0, 0)))
    h = f(out_p, src.reshape(NS, NCH, CHE), dst.reshape(NS, NCH, CHE))
    return h[:N]


@jax.jit
def kernel(x, edge_index, W1, W2):
    out = _mlp(x, W1, W2)
    h = _appnp(out, edge_index[0], edge_index[1])
    return _log_softmax(h)


# speedup vs baseline: 1.0107x; 1.0107x over previous
import jax
import jax.numpy as jnp
from jax import lax
from jax.experimental import pallas as pl
from jax.experimental.pallas import tpu as pltpu
from jax.experimental.pallas import tpu_sc as plsc

N = 10000
E = 320000
F_IN = 128
HID = 64
C = 16
K_HOPS = 10
ALPHA = 0.1

NS = 16
EPT = E // NS
CHE = 125
NCH = EPT // CHE
NP = 10112
RPT = NP // NS


def _mlp_body(x_ref, w1_ref, w2_ref, o_ref):
    h = lax.dot_general(x_ref[...], w1_ref[...], (((1,), (1,)), ((), ())),
                        preferred_element_type=jnp.float32)
    h = jnp.maximum(h, 0.0)
    o_ref[...] = lax.dot_general(h, w2_ref[...], (((1,), (1,)), ((), ())),
                                 preferred_element_type=jnp.float32)


def _mlp(x, W1, W2):
    blk = 2000
    return pl.pallas_call(
        _mlp_body,
        grid=(N // blk,),
        in_specs=[
            pl.BlockSpec((blk, F_IN), lambda i: (i, 0)),
            pl.BlockSpec((HID, F_IN), lambda i: (0, 0)),
            pl.BlockSpec((C, HID), lambda i: (0, 0)),
        ],
        out_specs=pl.BlockSpec((blk, C), lambda i: (i, 0)),
        out_shape=jax.ShapeDtypeStruct((N, C), jnp.float32),
    )(x, W1, W2)


def _lsm_body(h_ref, o_ref):
    h = h_ref[...]
    m = jnp.max(h, axis=1, keepdims=True)
    e = jnp.exp(h - m)
    lse = jnp.log(jnp.sum(e, axis=1, keepdims=True))
    o_ref[...] = h - m - lse


def _log_softmax(h):
    blk = 2000
    return pl.pallas_call(
        _lsm_body,
        grid=(N // blk,),
        in_specs=[pl.BlockSpec((blk, C), lambda i: (i, 0))],
        out_specs=pl.BlockSpec((blk, C), lambda i: (i, 0)),
        out_shape=jax.ShapeDtypeStruct((N, C), jnp.float32),
    )(h)


def _rsqrt16(d):
    t = 0.5 * (1.0 + d)
    for _ in range(9):
        t = 0.5 * (t + d / t)
    return 1.0 / t


ZCH = RPT // 8


def _appnp_body(out_hbm, src_hbm, dst_hbm, h_hbm,
                g_sh, acc_sh, acc2_sh,
                src_v, dst_v, rb0, rb1, onesb, zb, abuf,
                c1_t, c2_t, dinv_t, g_t,
                gs0, gs1):
    rowbufs = (rb0, rb1)
    gsems = (gs0, gs1)
    sid = lax.axis_index("s")
    cid = lax.axis_index("c")
    rbase = sid * RPT
    rows = pl.ds(rbase, RPT)

    pltpu.sync_copy(src_hbm.at[sid], src_v)
    pltpu.sync_copy(dst_hbm.at[sid], dst_v)

    def _fill1(r, _):
        onesb[r] = jnp.full((C,), 1.0, jnp.float32)
        return _
    lax.fori_loop(0, CHE, _fill1, 0)

    def _fill0(r, _):
        zb[r] = jnp.zeros((C,), jnp.float32)
        return _
    lax.fori_loop(0, ZCH, _fill0, 0)

    for z in range(8):
        pltpu.sync_copy(onesb.at[pl.ds(0, ZCH)],
                        acc_sh.at[pl.ds(rbase + z * ZCH, ZCH)])
        pltpu.sync_copy(zb, acc2_sh.at[pl.ds(rbase + z * ZCH, ZCH)])
    plsc.subcore_barrier()

    def _deg(j, _):
        pltpu.sync_copy(onesb, acc_sh.at[dst_v.at[j]], add=True)
        return _
    lax.fori_loop(0, NCH, _deg, 0)
    plsc.subcore_barrier()

    pltpu.sync_copy(acc_sh.at[rows], abuf)

    def _setup1(r, _):
        y = _rsqrt16(abuf[r])
        dinv_t[r] = y
        c1_t[r] = (1.0 - ALPHA) * y * y
        return _
    lax.fori_loop(0, RPT, _setup1, 0)
    for z in range(8):
        pltpu.sync_copy(zb, acc_sh.at[pl.ds(rbase + z * ZCH, ZCH)])

    pltpu.sync_copy(out_hbm.at[rows], abuf)

    def _setup2(r, _):
        g0 = dinv_t[r] * abuf[r]
        g_t[r] = g0
        c2_t[r] = ALPHA * g0
        return _
    lax.fori_loop(0, RPT, _setup2, 0)
    pltpu.sync_copy(g_t, g_sh.at[rows])
    plsc.subcore_barrier()

    rowbuf, rowbuf2 = rowbufs[0], rowbufs[1]
    gsA, gsB = gsems[0], gsems[1]

    def _hop(k, _):
        pltpu.async_copy(g_sh.at[src_v.at[0]], rowbuf, gsA)

        def _pair(i, _):
            j0 = 2 * i

            def _scat(buf, j):
                @pl.when(sid < NS // 2)
                def _lo():
                    pltpu.sync_copy(buf, acc_sh.at[dst_v.at[j]], add=True)

                @pl.when(sid >= NS // 2)
                def _hi():
                    pltpu.sync_copy(buf, acc2_sh.at[dst_v.at[j]], add=True)

            pltpu.make_async_copy(g_sh.at[src_v.at[j0]], rowbuf, gsA).wait()
            pltpu.async_copy(g_sh.at[src_v.at[j0 + 1]], rowbuf2, gsB)
            _scat(rowbuf, j0)
            jn = jnp.minimum(j0 + 2, NCH - 1)
            pltpu.make_async_copy(g_sh.at[src_v.at[j0 + 1]], rowbuf2,
                                  gsB).wait()
            pltpu.async_copy(g_sh.at[src_v.at[jn]], rowbuf, gsA)
            _scat(rowbuf2, j0 + 1)
            return _
        lax.fori_loop(0, NCH // 2, _pair, 0)
        pltpu.make_async_copy(g_sh.at[src_v.at[0]], rowbuf, gsA).wait()
        plsc.subcore_barrier()

        pltpu.sync_copy(acc_sh.at[rows], abuf)
        for z in range(8):
            pltpu.sync_copy(zb, acc_sh.at[pl.ds(rbase + z * ZCH, ZCH)])

        def _upd1(r, _):
            g_t[r] = abuf[r] + g_t[r]
            return _
        lax.fori_loop(0, RPT, _upd1, 0)

        pltpu.sync_copy(acc2_sh.at[rows], abuf)
        for z in range(8):
            pltpu.sync_copy(zb, acc2_sh.at[pl.ds(rbase + z * ZCH, ZCH)])

        def _upd2(r, _):
            g_t[r] = c1_t[r] * (abuf[r] + g_t[r]) + c2_t[r]
            return _
        lax.fori_loop(0, RPT, _upd2, 0)
        pltpu.sync_copy(g_t, g_sh.at[rows])
        plsc.subcore_barrier()
        return _
    lax.fori_loop(0, K_HOPS, _hop, 0)

    @pl.when(cid == 0)
    def _emit():
        def _div(r, _):
            abuf[r] = g_t[r] / dinv_t[r]
            return _
        lax.fori_loop(0, RPT, _div, 0)
        pltpu.sync_copy(abuf, h_hbm.at[rows])


def _appnp(out, src, dst):
    mesh = plsc.VectorSubcoreMesh(core_axis_name="c", subcore_axis_name="s")
    f = pl.kernel(
        _appnp_body,
        out_type=jax.ShapeDtypeStruct((NP, C), jnp.float32),
        mesh=mesh,
        compiler_params=pltpu.CompilerParams(use_tc_tiling_on_sc=False),
        scratch_types=[
            pltpu.VMEM_SHARED((NP, C), jnp.float32),
            pltpu.VMEM_SHARED((NP, C), jnp.float32),
            pltpu.VMEM_SHARED((NP, C), jnp.float32),
            pltpu.VMEM((NCH, CHE), jnp.int32),
            pltpu.VMEM((NCH, CHE), jnp.int32),
            pltpu.VMEM((CHE, C), jnp.float32),
            pltpu.VMEM((CHE, C), jnp.float32),
            pltpu.VMEM((CHE, C), jnp.float32),
            pltpu.VMEM((ZCH, C), jnp.float32),
            pltpu.VMEM((RPT, C), jnp.float32),
            pltpu.VMEM((RPT, C), jnp.float32),
            pltpu.VMEM((RPT, C), jnp.float32),
            pltpu.VMEM((RPT, C), jnp.float32),
            pltpu.VMEM((RPT, C), jnp.float32),
        ] + [pltpu.SemaphoreType.DMA] * 2,
    )
    out_p = jnp.pad(out, ((0, NP - N), (0, 0)))
    h = f(out_p, src.reshape(NS, NCH, CHE), dst.reshape(NS, NCH, CHE))
    return h[:N]


@jax.jit
def kernel(x, edge_index, W1, W2):
    out = _mlp(x, W1, W2)
    h = _appnp(out, edge_index[0], edge_index[1])
    return _log_softmax(h)


# speedup vs baseline: 1.1215x; 1.1097x over previous
import jax
import jax.numpy as jnp
from jax import lax
from jax.experimental import pallas as pl
from jax.experimental.pallas import tpu as pltpu
from jax.experimental.pallas import tpu_sc as plsc

N = 10000
E = 320000
F_IN = 128
HID = 64
C = 16
K_HOPS = 10
ALPHA = 0.1

NS = 16
EPT = E // NS
CHE = 125
NCH = EPT // CHE
NP = 10112
RPT = NP // NS


def _mlp_body(x_ref, w1_ref, w2_ref, o_ref):
    h = lax.dot_general(x_ref[...], w1_ref[...], (((1,), (1,)), ((), ())),
                        preferred_element_type=jnp.float32)
    h = jnp.maximum(h, 0.0)
    o_ref[...] = lax.dot_general(h, w2_ref[...], (((1,), (1,)), ((), ())),
                                 preferred_element_type=jnp.float32)


def _mlp(x, W1, W2):
    blk = 2000
    return pl.pallas_call(
        _mlp_body,
        grid=(N // blk,),
        in_specs=[
            pl.BlockSpec((blk, F_IN), lambda i: (i, 0)),
            pl.BlockSpec((HID, F_IN), lambda i: (0, 0)),
            pl.BlockSpec((C, HID), lambda i: (0, 0)),
        ],
        out_specs=pl.BlockSpec((blk, C), lambda i: (i, 0)),
        out_shape=jax.ShapeDtypeStruct((N, C), jnp.float32),
    )(x, W1, W2)


def _lsm_body(h_ref, o_ref):
    h = h_ref[...]
    m = jnp.max(h, axis=1, keepdims=True)
    e = jnp.exp(h - m)
    lse = jnp.log(jnp.sum(e, axis=1, keepdims=True))
    o_ref[...] = h - m - lse


def _log_softmax(h):
    blk = 2000
    return pl.pallas_call(
        _lsm_body,
        grid=(N // blk,),
        in_specs=[pl.BlockSpec((blk, C), lambda i: (i, 0))],
        out_specs=pl.BlockSpec((blk, C), lambda i: (i, 0)),
        out_shape=jax.ShapeDtypeStruct((N, C), jnp.float32),
    )(h)


def _rsqrt16(d):
    t = 0.5 * (1.0 + d)
    for _ in range(9):
        t = 0.5 * (t + d / t)
    return 1.0 / t


ZCH = RPT // 8


def _appnp_body(out_hbm, src_hbm, dst_hbm, h_hbm,
                g_sh, acc_sh, acc2_sh,
                src_v, dst_v, rb0, rb1, onesb, zb, abuf,
                c1_t, c2_t, dinv_t, g_t,
                gs0, gs1):
    rowbufs = (rb0, rb1)
    gsems = (gs0, gs1)
    sid = lax.axis_index("s")
    cid = lax.axis_index("c")
    rbase = sid * RPT
    rows = pl.ds(rbase, RPT)

    pltpu.sync_copy(src_hbm.at[sid], src_v)
    pltpu.sync_copy(dst_hbm.at[sid], dst_v)

    @plsc.parallel_loop(0, CHE, 1, unroll=8)
    def _fill1(r):
        onesb[r] = jnp.full((C,), 1.0, jnp.float32)

    @plsc.parallel_loop(0, ZCH, 1, unroll=8)
    def _fill0(r):
        zb[r] = jnp.zeros((C,), jnp.float32)

    for z in range(8):
        pltpu.sync_copy(onesb.at[pl.ds(0, ZCH)],
                        acc_sh.at[pl.ds(rbase + z * ZCH, ZCH)])
        pltpu.sync_copy(zb, acc2_sh.at[pl.ds(rbase + z * ZCH, ZCH)])
    plsc.subcore_barrier()

    def _deg(j, _):
        pltpu.sync_copy(onesb, acc_sh.at[dst_v.at[j]], add=True)
        return _
    lax.fori_loop(0, NCH, _deg, 0)
    plsc.subcore_barrier()

    pltpu.sync_copy(acc_sh.at[rows], abuf)

    @plsc.parallel_loop(0, RPT, 1, unroll=4)
    def _setup1(r):
        y = _rsqrt16(abuf[r])
        dinv_t[r] = y
        c1_t[r] = (1.0 - ALPHA) * y * y
    for z in range(8):
        pltpu.sync_copy(zb, acc_sh.at[pl.ds(rbase + z * ZCH, ZCH)])

    pltpu.sync_copy(out_hbm.at[rows], abuf)

    @plsc.parallel_loop(0, RPT, 1, unroll=8)
    def _setup2(r):
        g0 = dinv_t[r] * abuf[r]
        g_t[r] = g0
        c2_t[r] = ALPHA * g0
    pltpu.sync_copy(g_t, g_sh.at[rows])
    plsc.subcore_barrier()

    rowbuf, rowbuf2 = rowbufs[0], rowbufs[1]
    gsA, gsB = gsems[0], gsems[1]

    def _hop(k, _):
        pltpu.async_copy(g_sh.at[src_v.at[0]], rowbuf, gsA)

        def _pair(i, _):
            j0 = 2 * i

            def _scat(buf, j):
                @pl.when(sid < NS // 2)
                def _lo():
                    pltpu.sync_copy(buf, acc_sh.at[dst_v.at[j]], add=True)

                @pl.when(sid >= NS // 2)
                def _hi():
                    pltpu.sync_copy(buf, acc2_sh.at[dst_v.at[j]], add=True)

            pltpu.make_async_copy(g_sh.at[src_v.at[j0]], rowbuf, gsA).wait()
            pltpu.async_copy(g_sh.at[src_v.at[j0 + 1]], rowbuf2, gsB)
            _scat(rowbuf, j0)
            jn = jnp.minimum(j0 + 2, NCH - 1)
            pltpu.make_async_copy(g_sh.at[src_v.at[j0 + 1]], rowbuf2,
                                  gsB).wait()
            pltpu.async_copy(g_sh.at[src_v.at[jn]], rowbuf, gsA)
            _scat(rowbuf2, j0 + 1)
            return _
        lax.fori_loop(0, NCH // 2, _pair, 0)
        pltpu.make_async_copy(g_sh.at[src_v.at[0]], rowbuf, gsA).wait()
        plsc.subcore_barrier()

        pltpu.sync_copy(acc_sh.at[rows], abuf)
        for z in range(8):
            pltpu.sync_copy(zb, acc_sh.at[pl.ds(rbase + z * ZCH, ZCH)])

        @plsc.parallel_loop(0, RPT, 1, unroll=8)
        def _upd1(r):
            g_t[r] = abuf[r] + g_t[r]

        pltpu.sync_copy(acc2_sh.at[rows], abuf)
        for z in range(8):
            pltpu.sync_copy(zb, acc2_sh.at[pl.ds(rbase + z * ZCH, ZCH)])

        @plsc.parallel_loop(0, RPT, 1, unroll=8)
        def _upd2(r):
            g_t[r] = c1_t[r] * (abuf[r] + g_t[r]) + c2_t[r]
        pltpu.sync_copy(g_t, g_sh.at[rows])
        plsc.subcore_barrier()
        return _
    lax.fori_loop(0, K_HOPS, _hop, 0)

    @pl.when(cid == 0)
    def _emit():
        @plsc.parallel_loop(0, RPT, 1, unroll=8)
        def _div(r):
            abuf[r] = g_t[r] / dinv_t[r]
        pltpu.sync_copy(abuf, h_hbm.at[rows])


def _appnp(out, src, dst):
    mesh = plsc.VectorSubcoreMesh(core_axis_name="c", subcore_axis_name="s")
    f = pl.kernel(
        _appnp_body,
        out_type=jax.ShapeDtypeStruct((NP, C), jnp.float32),
        mesh=mesh,
        compiler_params=pltpu.CompilerParams(use_tc_tiling_on_sc=False),
        scratch_types=[
            pltpu.VMEM_SHARED((NP, C), jnp.float32),
            pltpu.VMEM_SHARED((NP, C), jnp.float32),
            pltpu.VMEM_SHARED((NP, C), jnp.float32),
            pltpu.VMEM((NCH, CHE), jnp.int32),
            pltpu.VMEM((NCH, CHE), jnp.int32),
            pltpu.VMEM((CHE, C), jnp.float32),
            pltpu.VMEM((CHE, C), jnp.float32),
            pltpu.VMEM((CHE, C), jnp.float32),
            pltpu.VMEM((ZCH, C), jnp.float32),
            pltpu.VMEM((RPT, C), jnp.float32),
            pltpu.VMEM((RPT, C), jnp.float32),
            pltpu.VMEM((RPT, C), jnp.float32),
            pltpu.VMEM((RPT, C), jnp.float32),
            pltpu.VMEM((RPT, C), jnp.float32),
        ] + [pltpu.SemaphoreType.DMA] * 2,
    )
    out_p = jnp.pad(out, ((0, NP - N), (0, 0)))
    h = f(out_p, src.reshape(NS, NCH, CHE), dst.reshape(NS, NCH, CHE))
    return h[:N]


@jax.jit
def kernel(x, edge_index, W1, W2):
    out = _mlp(x, W1, W2)
    h = _appnp(out, edge_index[0], edge_index[1])
    return _log_softmax(h)
